# speedup vs baseline: 18.5914x; 18.5914x over previous
import jax
import jax.numpy as jnp
from jax.experimental import pallas as pl
from jax.experimental.pallas import tpu as pltpu

_K = 256
_N = 32768
_ROWS = 64
_BLOCK_ROWS = 8


def _topk_mask_body(x_ref, o_ref):
    x = x_ref[...]
    i = pltpu.bitcast(x, jnp.int32)
    key = i ^ (jnp.right_shift(i, 31) & jnp.int32(0x7FFFFFFF))

    k = jnp.int32(_K)
    int_min = jnp.int32(-2147483648)

    cnt0 = jnp.sum((key >= 0).astype(jnp.int32), axis=1, keepdims=True)
    thr = jnp.where(cnt0 >= k, jnp.int32(0), int_min)

    def body(t, thr):
        b = jnp.int32(30) - t
        cand = thr | jnp.left_shift(jnp.int32(1), b)
        cnt = jnp.sum((key >= cand).astype(jnp.int32), axis=1, keepdims=True)
        return jnp.where(cnt >= k, cand, thr)

    thr = jax.lax.fori_loop(0, 31, body, thr)
    o_ref[...] = (key >= thr).astype(jnp.float32)


def kernel(scores):
    return pl.pallas_call(
        _topk_mask_body,
        grid=(_ROWS // _BLOCK_ROWS,),
        in_specs=[pl.BlockSpec((_BLOCK_ROWS, _N), lambda i: (i, 0))],
        out_specs=pl.BlockSpec((_BLOCK_ROWS, _N), lambda i: (i, 0)),
        out_shape=jax.ShapeDtypeStruct((_ROWS, _N), jnp.float32),
    )(scores)
